# baseline (device time: 31821 ns/iter reference)
import jax
import jax.numpy as jnp
from jax import lax
from jax.experimental import pallas as pl
from jax.experimental.pallas import tpu as pltpu

N_DEV = 16
B, Sq, Skv, Dh = 2, 128, 128, 64
HQ_LOCAL = 4
D_MODEL = 512
CH = Sq // N_DEV


def kernel(x, Wq, K_ext, V_ext, Wo):
    my = lax.axis_index("i")
    K_loc = lax.dynamic_slice_in_dim(K_ext, my * HQ_LOCAL, HQ_LOCAL, axis=2)
    V_loc = lax.dynamic_slice_in_dim(V_ext, my * HQ_LOCAL, HQ_LOCAL, axis=2)

    def body(x_ref, wq_ref, k_ref, v_ref, wo_ref, out_ref,
             partial_buf, rs_buf, ag_buf,
             rs_send, rs_recv, ag_send, ag_recv):
        my_pos = lax.axis_index("i")

        xv = x_ref[...].reshape(B * Sq, D_MODEL)
        q = jnp.dot(xv, wq_ref[...], preferred_element_type=jnp.float32)

        qb = lax.broadcasted_iota(jnp.int32, (Sq, Skv), 0) // 64
        kb = lax.broadcasted_iota(jnp.int32, (Sq, Skv), 1) // 64
        mask = (qb == kb) | (kb == 0) | ((qb + kb) % 3 == 0)

        ctx_rows = []
        for b in range(B):
            heads = []
            for h in range(HQ_LOCAL):
                qbh = q[b * Sq:(b + 1) * Sq, h * Dh:(h + 1) * Dh]
                kbh = k_ref[b, :, h, :]
                vbh = v_ref[b, :, h, :]
                s = jnp.dot(qbh, kbh.T, preferred_element_type=jnp.float32)
                s = s * 0.125
                s = jnp.where(mask, s, -1e9)
                m = jnp.max(s, axis=-1, keepdims=True)
                w = jnp.exp(s - m)
                w = w / jnp.sum(w, axis=-1, keepdims=True)
                heads.append(jnp.dot(w, vbh, preferred_element_type=jnp.float32))
            ctx_rows.append(jnp.concatenate(heads, axis=1))
        ctx = jnp.concatenate(ctx_rows, axis=0)

        partial = jnp.dot(ctx, wo_ref[...], preferred_element_type=jnp.float32)
        partial = partial.reshape(B, Sq, D_MODEL)
        for j in range(N_DEV):
            partial_buf[j] = partial[:, j * CH:(j + 1) * CH, :]

        rs_sends = []
        for d in range(1, N_DEV):
            j = (my_pos + d) % N_DEV
            rdma = pltpu.make_async_remote_copy(
                src_ref=partial_buf.at[j],
                dst_ref=rs_buf.at[my_pos],
                send_sem=rs_send.at[j],
                recv_sem=rs_recv.at[my_pos],
                device_id=(j,),
                device_id_type=pl.DeviceIdType.MESH,
            )
            rdma.start()
            rs_sends.append(rdma)

        rs_buf[my_pos] = partial_buf[my_pos]

        for d in range(1, N_DEV):
            j = (my_pos + d) % N_DEV
            pltpu.make_async_remote_copy(
                src_ref=partial_buf.at[j],
                dst_ref=rs_buf.at[j],
                send_sem=rs_send.at[j],
                recv_sem=rs_recv.at[j],
                device_id=(j,),
                device_id_type=pl.DeviceIdType.MESH,
            ).wait_recv()

        reduced = jnp.sum(rs_buf[...], axis=0)
        ag_buf[my_pos] = reduced

        ag_sends = []
        for d in range(1, N_DEV):
            j = (my_pos + d) % N_DEV
            rdma = pltpu.make_async_remote_copy(
                src_ref=ag_buf.at[my_pos],
                dst_ref=ag_buf.at[my_pos],
                send_sem=ag_send.at[j],
                recv_sem=ag_recv.at[my_pos],
                device_id=(j,),
                device_id_type=pl.DeviceIdType.MESH,
            )
            rdma.start()
            ag_sends.append(rdma)

        for rdma in rs_sends:
            rdma.wait_send()

        for d in range(1, N_DEV):
            j = (my_pos + d) % N_DEV
            pltpu.make_async_remote_copy(
                src_ref=ag_buf.at[my_pos],
                dst_ref=ag_buf.at[j],
                send_sem=ag_send.at[j],
                recv_sem=ag_recv.at[j],
                device_id=(j,),
                device_id_type=pl.DeviceIdType.MESH,
            ).wait_recv()

        for j in range(N_DEV):
            out_ref[:, j * CH:(j + 1) * CH, :] = ag_buf[j]

        for rdma in ag_sends:
            rdma.wait_send()

    return pl.pallas_call(
        body,
        out_shape=jax.ShapeDtypeStruct((B, Sq, D_MODEL), jnp.float32),
        in_specs=[pl.BlockSpec(memory_space=pltpu.VMEM)] * 5,
        out_specs=pl.BlockSpec(memory_space=pltpu.VMEM),
        scratch_shapes=[
            pltpu.VMEM((N_DEV, B, CH, D_MODEL), jnp.float32),
            pltpu.VMEM((N_DEV, B, CH, D_MODEL), jnp.float32),
            pltpu.VMEM((N_DEV, B, CH, D_MODEL), jnp.float32),
            pltpu.SemaphoreType.DMA((N_DEV,)),
            pltpu.SemaphoreType.DMA((N_DEV,)),
            pltpu.SemaphoreType.DMA((N_DEV,)),
            pltpu.SemaphoreType.DMA((N_DEV,)),
        ],
    )(x, Wq, K_loc, V_loc, Wo)


# device time: 31480 ns/iter; 1.0108x vs baseline; 1.0108x over previous
import jax
import jax.numpy as jnp
from jax import lax
from jax.experimental import pallas as pl
from jax.experimental.pallas import tpu as pltpu

N_DEV = 16
B, Sq, Skv, Dh = 2, 128, 128, 64
HQ_LOCAL = 4
D_MODEL = 512
CH = Sq // N_DEV


def kernel(x, Wq, K_ext, V_ext, Wo):
    my = lax.axis_index("i")
    K_loc = lax.dynamic_slice_in_dim(K_ext, my * HQ_LOCAL, HQ_LOCAL, axis=2)
    V_loc = lax.dynamic_slice_in_dim(V_ext, my * HQ_LOCAL, HQ_LOCAL, axis=2)

    def body(x_ref, wq_ref, k_ref, v_ref, wo_ref, out_ref,
             partial_buf, rs_buf, ag_src,
             rs_send, rs_recv, ag_send, ag_recv):
        my_pos = lax.axis_index("i")

        xv = x_ref[...].reshape(B * Sq, D_MODEL)
        q = jnp.dot(xv, wq_ref[...], preferred_element_type=jnp.float32)

        qb = lax.broadcasted_iota(jnp.int32, (Sq, Skv), 0) // 64
        kb = lax.broadcasted_iota(jnp.int32, (Sq, Skv), 1) // 64
        mask = (qb == kb) | (kb == 0) | ((qb + kb) % 3 == 0)

        ctx_rows = []
        for b in range(B):
            heads = []
            for h in range(HQ_LOCAL):
                qbh = q[b * Sq:(b + 1) * Sq, h * Dh:(h + 1) * Dh]
                kbh = k_ref[b, :, h, :]
                vbh = v_ref[b, :, h, :]
                s = jnp.dot(qbh, kbh.T, preferred_element_type=jnp.float32)
                s = s * 0.125
                s = jnp.where(mask, s, -1e9)
                m = jnp.max(s, axis=-1, keepdims=True)
                w = jnp.exp(s - m)
                w = w / jnp.sum(w, axis=-1, keepdims=True)
                heads.append(jnp.dot(w, vbh, preferred_element_type=jnp.float32))
            ctx_rows.append(jnp.concatenate(heads, axis=1))
        ctx = jnp.concatenate(ctx_rows, axis=0)

        partial = jnp.dot(ctx, wo_ref[...], preferred_element_type=jnp.float32)
        partial = partial.reshape(B, Sq, D_MODEL)
        for j in range(N_DEV):
            partial_buf[j] = partial[:, j * CH:(j + 1) * CH, :]

        rs_sends = []
        for d in range(1, N_DEV):
            j = (my_pos + d) % N_DEV
            rdma = pltpu.make_async_remote_copy(
                src_ref=partial_buf.at[j],
                dst_ref=rs_buf.at[my_pos],
                send_sem=rs_send.at[j],
                recv_sem=rs_recv.at[my_pos],
                device_id=(j,),
                device_id_type=pl.DeviceIdType.MESH,
            )
            rdma.start()
            rs_sends.append(rdma)

        rs_buf[my_pos] = partial_buf[my_pos]

        for d in range(1, N_DEV):
            j = (my_pos + d) % N_DEV
            pltpu.make_async_remote_copy(
                src_ref=partial_buf.at[j],
                dst_ref=rs_buf.at[j],
                send_sem=rs_send.at[j],
                recv_sem=rs_recv.at[j],
                device_id=(j,),
                device_id_type=pl.DeviceIdType.MESH,
            ).wait_recv()

        reduced = jnp.sum(rs_buf[...], axis=0)
        ag_src[...] = reduced
        out_ref[:, pl.ds(my_pos * CH, CH), :] = reduced

        ag_sends = []
        for d in range(1, N_DEV):
            j = (my_pos + d) % N_DEV
            rdma = pltpu.make_async_remote_copy(
                src_ref=ag_src,
                dst_ref=out_ref.at[:, pl.ds(my_pos * CH, CH), :],
                send_sem=ag_send.at[j],
                recv_sem=ag_recv.at[my_pos],
                device_id=(j,),
                device_id_type=pl.DeviceIdType.MESH,
            )
            rdma.start()
            ag_sends.append(rdma)

        for rdma in rs_sends:
            rdma.wait_send()

        for d in range(1, N_DEV):
            j = (my_pos + d) % N_DEV
            pltpu.make_async_remote_copy(
                src_ref=ag_src,
                dst_ref=out_ref.at[:, pl.ds(j * CH, CH), :],
                send_sem=ag_send.at[j],
                recv_sem=ag_recv.at[j],
                device_id=(j,),
                device_id_type=pl.DeviceIdType.MESH,
            ).wait_recv()

        for rdma in ag_sends:
            rdma.wait_send()

    return pl.pallas_call(
        body,
        out_shape=jax.ShapeDtypeStruct((B, Sq, D_MODEL), jnp.float32),
        in_specs=[pl.BlockSpec(memory_space=pltpu.VMEM)] * 5,
        out_specs=pl.BlockSpec(memory_space=pltpu.VMEM),
        scratch_shapes=[
            pltpu.VMEM((N_DEV, B, CH, D_MODEL), jnp.float32),
            pltpu.VMEM((N_DEV, B, CH, D_MODEL), jnp.float32),
            pltpu.VMEM((B, CH, D_MODEL), jnp.float32),
            pltpu.SemaphoreType.DMA((N_DEV,)),
            pltpu.SemaphoreType.DMA((N_DEV,)),
            pltpu.SemaphoreType.DMA((N_DEV,)),
            pltpu.SemaphoreType.DMA((N_DEV,)),
        ],
    )(x, Wq, K_loc, V_loc, Wo)


# device time: 20962 ns/iter; 1.5180x vs baseline; 1.5018x over previous
import jax
import jax.numpy as jnp
from jax import lax
from jax.experimental import pallas as pl
from jax.experimental.pallas import tpu as pltpu

N_DEV = 16
B, Sq, Skv, Dh = 2, 128, 128, 64
HQ_LOCAL = 4
D_MODEL = 512
ROWS = B * Sq
CH = ROWS // N_DEV

COMM_DT = jnp.bfloat16
MM_DT = jnp.bfloat16


def kernel(x, Wq, K_ext, V_ext, Wo):
    my = lax.axis_index("i")
    K_loc = lax.dynamic_slice_in_dim(K_ext, my * HQ_LOCAL, HQ_LOCAL, axis=2)
    V_loc = lax.dynamic_slice_in_dim(V_ext, my * HQ_LOCAL, HQ_LOCAL, axis=2)

    def body(x_ref, wq_ref, k_loc, v_loc, wo_ref, out_ref,
             partial_buf, rs_buf, ag_src, ag_dst,
             rs_send, rs_recv, ag_send, ag_recv):
        my_pos = lax.axis_index("i")

        barrier_sem = pltpu.get_barrier_semaphore()
        for d in range(1, N_DEV):
            j = (my_pos + d) % N_DEV
            pl.semaphore_signal(
                barrier_sem, inc=1,
                device_id=(j,), device_id_type=pl.DeviceIdType.MESH,
            )

        xv = x_ref[...].reshape(ROWS, D_MODEL)
        q = jnp.dot(xv.astype(MM_DT), wq_ref[...].astype(MM_DT),
                    preferred_element_type=jnp.float32)

        qb = lax.broadcasted_iota(jnp.int32, (Sq, Skv), 0) // 64
        kb = lax.broadcasted_iota(jnp.int32, (Sq, Skv), 1) // 64
        mask = (qb == kb) | (kb == 0) | ((qb + kb) % 3 == 0)

        ctx_rows = []
        for b in range(B):
            heads = []
            for h in range(HQ_LOCAL):
                qbh = q[b * Sq:(b + 1) * Sq, h * Dh:(h + 1) * Dh]
                kbh = k_loc[b, :, h, :]
                vbh = v_loc[b, :, h, :]
                s = jnp.dot(qbh.astype(MM_DT), kbh.astype(MM_DT).T,
                            preferred_element_type=jnp.float32)
                s = s * 0.125
                s = jnp.where(mask, s, -1e9)
                m = jnp.max(s, axis=-1, keepdims=True)
                w = jnp.exp(s - m)
                w = w / jnp.sum(w, axis=-1, keepdims=True)
                heads.append(jnp.dot(w.astype(MM_DT), vbh.astype(MM_DT),
                                     preferred_element_type=jnp.float32))
            ctx_rows.append(jnp.concatenate(heads, axis=1))
        ctx = jnp.concatenate(ctx_rows, axis=0)

        partial = jnp.dot(ctx.astype(MM_DT), wo_ref[...].astype(MM_DT),
                          preferred_element_type=jnp.float32)
        partial = partial.astype(COMM_DT)
        for j in range(N_DEV):
            partial_buf[j] = partial[j * CH:(j + 1) * CH, :]

        pl.semaphore_wait(barrier_sem, N_DEV - 1)

        rs_sends = []
        for d in range(1, N_DEV):
            j = (my_pos + d) % N_DEV
            rdma = pltpu.make_async_remote_copy(
                src_ref=partial_buf.at[j],
                dst_ref=rs_buf.at[my_pos],
                send_sem=rs_send.at[j],
                recv_sem=rs_recv.at[my_pos],
                device_id=(j,),
                device_id_type=pl.DeviceIdType.MESH,
            )
            rdma.start()
            rs_sends.append(rdma)

        rs_buf[my_pos] = partial_buf[my_pos]

        for d in range(1, N_DEV):
            j = (my_pos + d) % N_DEV
            pltpu.make_async_remote_copy(
                src_ref=partial_buf.at[j],
                dst_ref=rs_buf.at[j],
                send_sem=rs_send.at[j],
                recv_sem=rs_recv.at[j],
                device_id=(j,),
                device_id_type=pl.DeviceIdType.MESH,
            ).wait_recv()

        reduced = jnp.sum(rs_buf[...].astype(jnp.float32), axis=0)
        ag_src[...] = reduced.astype(COMM_DT)
        ag_dst[my_pos] = ag_src[...]

        ag_sends = []
        for d in range(1, N_DEV):
            j = (my_pos + d) % N_DEV
            rdma = pltpu.make_async_remote_copy(
                src_ref=ag_src,
                dst_ref=ag_dst.at[my_pos],
                send_sem=ag_send.at[j],
                recv_sem=ag_recv.at[my_pos],
                device_id=(j,),
                device_id_type=pl.DeviceIdType.MESH,
            )
            rdma.start()
            ag_sends.append(rdma)

        for rdma in rs_sends:
            rdma.wait_send()

        for d in range(1, N_DEV):
            j = (my_pos + d) % N_DEV
            pltpu.make_async_remote_copy(
                src_ref=ag_src,
                dst_ref=ag_dst.at[j],
                send_sem=ag_send.at[j],
                recv_sem=ag_recv.at[j],
                device_id=(j,),
                device_id_type=pl.DeviceIdType.MESH,
            ).wait_recv()

        full = ag_dst[...].reshape(ROWS, D_MODEL).astype(jnp.float32)
        out_ref[...] = full.reshape(B, Sq, D_MODEL)

        for rdma in ag_sends:
            rdma.wait_send()

    return pl.pallas_call(
        body,
        out_shape=jax.ShapeDtypeStruct((B, Sq, D_MODEL), jnp.float32),
        in_specs=[
            pl.BlockSpec(memory_space=pltpu.MemorySpace.VMEM),
            pl.BlockSpec(memory_space=pltpu.MemorySpace.VMEM),
            pl.BlockSpec(memory_space=pltpu.MemorySpace.VMEM),
            pl.BlockSpec(memory_space=pltpu.MemorySpace.VMEM),
            pl.BlockSpec(memory_space=pltpu.MemorySpace.VMEM),
        ],
        out_specs=pl.BlockSpec(memory_space=pltpu.MemorySpace.VMEM),
        scratch_shapes=[
            pltpu.VMEM((N_DEV, CH, D_MODEL), COMM_DT),
            pltpu.VMEM((N_DEV, CH, D_MODEL), COMM_DT),
            pltpu.VMEM((CH, D_MODEL), COMM_DT),
            pltpu.VMEM((N_DEV, CH, D_MODEL), COMM_DT),
            pltpu.SemaphoreType.DMA((N_DEV,)),
            pltpu.SemaphoreType.DMA((N_DEV,)),
            pltpu.SemaphoreType.DMA((N_DEV,)),
            pltpu.SemaphoreType.DMA((N_DEV,)),
        ],
        compiler_params=pltpu.CompilerParams(collective_id=0),
    )(x, Wq, K_loc, V_loc, Wo)
